# baseline (speedup 1.0000x reference)
import functools

import jax
import jax.numpy as jnp
from jax import lax
from jax.experimental import pallas as pl
from jax.experimental.pallas import tpu as pltpu
from jax.experimental.pallas import tpu_sc as plsc

N_NODES = 10000
D = 128
DH = 64
E_EDGES = 320000

NUM_CORES = 2
NUM_SUBCORES = 16
NW = NUM_CORES * NUM_SUBCORES

CHUNK = 128
CHUNKS = 160
NBUF = 2
BI = 16
NBLOCKS = CHUNKS // BI
NGROUPS = BI // NBUF
E_PAD = NUM_SUBCORES * CHUNKS * CHUNK
DCHUNKS = E_PAD // (NW * CHUNK)
N_PAD = 10240
SL = N_PAD // NUM_SUBCORES
NSCALE = SL // 16
ZROWS = 16

_mesh = plsc.VectorSubcoreMesh(
    core_axis_name="c", subcore_axis_name="s",
    num_cores=NUM_CORES, num_subcores=NUM_SUBCORES)


@functools.partial(
    pl.kernel,
    out_type=jax.ShapeDtypeStruct((NUM_CORES * 2 * N_PAD,), jnp.float32),
    mesh=_mesh,
    scratch_types=[
        pltpu.VMEM((2 * DCHUNKS, CHUNK), jnp.int32),
        pltpu.VMEM((CHUNK,), jnp.float32),
        pltpu.VMEM((2 * N_PAD // NUM_SUBCORES,), jnp.float32),
        pltpu.VMEM_SHARED((2 * N_PAD,), jnp.float32),
        pltpu.SemaphoreType.DMA,
    ],
)
def _sc_degrees(idx_hbm, out_hbm, idx_v, ones_v, z_v, acc, sem):
    c = lax.axis_index("c")
    s = lax.axis_index("s")
    wid = s * NUM_CORES + c

    ones16 = jnp.ones((16,), jnp.float32)
    zero16 = jnp.zeros((16,), jnp.float32)
    for i in range(CHUNK // 16):
        ones_v[pl.ds(i * 16, 16)] = ones16

    zlen = 2 * N_PAD // NUM_SUBCORES

    @pl.loop(0, zlen // 16)
    def _(i):
        z_v[pl.ds(i * 16, 16)] = zero16

    pltpu.sync_copy(z_v, acc.at[pl.ds(s * zlen, zlen)])
    plsc.subcore_barrier()

    pltpu.async_copy(idx_hbm.at[wid], idx_v, sem).wait()

    @pl.loop(0, 2 * DCHUNKS)
    def _(j):
        pltpu.sync_copy(ones_v, acc.at[idx_v.at[j]], add=True)

    plsc.subcore_barrier()
    pltpu.sync_copy(acc.at[pl.ds(s * zlen, zlen)],
                    out_hbm.at[pl.ds(c * 2 * N_PAD + s * zlen, zlen)])


@functools.partial(
    pl.kernel,
    out_type=[jax.ShapeDtypeStruct((NUM_CORES, N_PAD, DH), jnp.float32),
              jax.ShapeDtypeStruct((2 * N_PAD,), jnp.float32)],
    mesh=_mesh,
    scratch_types=[
        pltpu.VMEM((2, BI, CHUNK), jnp.int32),
        pltpu.VMEM((2, BI, CHUNK), jnp.int32),
        [pltpu.VMEM((CHUNK, DH), jnp.float32)] * NBUF,
        pltpu.VMEM((ZROWS, DH), jnp.float32),
        pltpu.VMEM((SL,), jnp.float32),
        pltpu.VMEM((16, DH), jnp.float32),
        pltpu.VMEM((N_PAD,), jnp.float32),
        pltpu.VMEM((SL,), jnp.float32),
        pltpu.VMEM((SL,), jnp.float32),
        pltpu.VMEM((SL,), jnp.float32),
        pltpu.VMEM((CHUNK,), jnp.float32),
        pltpu.VMEM_SHARED((N_PAD, DH), jnp.float32),
        pltpu.VMEM_SHARED((N_PAD, DH), jnp.float32),
        pltpu.VMEM_SHARED((N_PAD,), jnp.float32),
        pltpu.VMEM_SHARED((N_PAD,), jnp.float32),
        pltpu.VMEM_SHARED((N_PAD,), jnp.float32),
        [pltpu.SemaphoreType.DMA] * NBUF,
        [pltpu.SemaphoreType.DMA] * NBUF,
        [pltpu.SemaphoreType.DMA] * 2,
        [pltpu.SemaphoreType.DMA] * 2,
    ],
    compiler_params=pltpu.CompilerParams(use_tc_tiling_on_sc=False,
                                         needs_layout_passes=False),
)
def _sc_gcn3(ga_hbm, gb_hbm, no_hbm, d_hbm, ni_hbm, src_hbm, dst_hbm,
             y_hbm, qv_hbm,
             src_v, dst_v, rows, z_v, zv1, sbuf, t_v, tb_v, dv_v, niv_v,
             sval, gtab, acc, vacc1, vacc2, t2s, gsem, ssem, bs_sem, bd_sem):
    c = lax.axis_index("c")
    s = lax.axis_index("s")
    zero16 = jnp.zeros((16,), jnp.float32)

    @pl.loop(0, ZROWS)
    def _(r):
        for i in range(DH // 16):
            z_v[r, pl.ds(i * 16, 16)] = zero16

    @pl.loop(0, NSCALE)
    def _(p):
        zv1[pl.ds(p * 16, 16)] = zero16

    sl = pl.ds(s * SL, SL)

    @pl.loop(0, SL // ZROWS)
    def _(k):
        pltpu.sync_copy(z_v, acc.at[pl.ds(s * SL + k * ZROWS, ZROWS)])

    pltpu.sync_copy(zv1, vacc1.at[sl])
    pltpu.sync_copy(zv1, vacc2.at[sl])

    @pl.when(c == 0)
    def _():
        pltpu.sync_copy(ga_hbm.at[sl], gtab.at[sl])

    @pl.when(c == 1)
    def _():
        pltpu.sync_copy(gb_hbm.at[sl], gtab.at[sl])

    pltpu.sync_copy(no_hbm, t_v)
    pltpu.sync_copy(d_hbm.at[sl], dv_v)
    pltpu.sync_copy(ni_hbm.at[sl], niv_v)

    pltpu.async_copy(src_hbm.at[s, pl.ds(0, BI)], src_v.at[0], bs_sem[0])
    pltpu.async_copy(dst_hbm.at[s, pl.ds(0, BI)], dst_v.at[0], bd_sem[0])
    plsc.subcore_barrier()

    for it in (1, 2, 3):
        vacc_it = vacc1 if it == 1 else vacc2
        for k in range(NBLOCKS):
            slot = k % 2
            pltpu.make_async_copy(src_hbm.at[s, pl.ds(k * BI, BI)],
                                  src_v.at[slot], bs_sem[slot]).wait()
            pltpu.make_async_copy(dst_hbm.at[s, pl.ds(k * BI, BI)],
                                  dst_v.at[slot], bd_sem[slot]).wait()
            if k + 1 < NBLOCKS:
                nslot = (k + 1) % 2
                pltpu.async_copy(src_hbm.at[s, pl.ds((k + 1) * BI, BI)],
                                 src_v.at[nslot], bs_sem[nslot])
                pltpu.async_copy(dst_hbm.at[s, pl.ds((k + 1) * BI, BI)],
                                 dst_v.at[nslot], bd_sem[nslot])

            for b in range(NBUF):
                pltpu.async_copy(gtab.at[src_v.at[slot, b]], rows[b],
                                 gsem[b])

            @pl.loop(0, NGROUPS)
            def _(t, slot=slot):
                base = t * NBUF
                for b in range(NBUF):
                    pltpu.make_async_copy(gtab.at[src_v.at[slot, base + b]],
                                          rows[b], gsem[b]).wait()
                    pltpu.async_copy(rows[b],
                                     acc.at[dst_v.at[slot, base + b]],
                                     ssem[b], add=True)
                for b in range(NBUF):
                    @pl.when(t + 1 < NGROUPS)
                    def _(b=b):
                        pltpu.make_async_copy(
                            rows[b], acc.at[dst_v.at[slot, base + b]],
                            ssem[b]).wait()
                        pltpu.async_copy(
                            gtab.at[src_v.at[slot, base + NBUF + b]],
                            rows[b], gsem[b])

            for b in range(NBUF):
                pltpu.make_async_copy(rows[b],
                                      acc.at[dst_v.at[slot, BI - NBUF + b]],
                                      ssem[b]).wait()

            if it < 3:
                @pl.loop(0, BI)
                def _(j, slot=slot, vacc_it=vacc_it):
                    for p in range(CHUNK // 16):
                        idx16 = src_v[slot, j, pl.ds(p * 16, 16)]
                        sval[pl.ds(p * 16, 16)] = plsc.load_gather(
                            t_v, [idx16])
                    pltpu.sync_copy(sval, vacc_it.at[dst_v.at[slot, j]],
                                    add=True)

        plsc.subcore_barrier()

        if it < 3:
            @pl.loop(0, NSCALE)
            def _(blk):
                r0 = s * SL + blk * 16
                pltpu.sync_copy(acc.at[pl.ds(r0, 16)], sbuf)
                d16 = dv_v[pl.ds(blk * 16, 16)]
                for i in range(16):
                    di = d16[i]
                    for q in range(DH // 16):
                        sbuf[i, pl.ds(q * 16, 16)] = (
                            sbuf[i, pl.ds(q * 16, 16)] * di)
                pltpu.sync_copy(sbuf, gtab.at[pl.ds(r0, 16)])
                pltpu.sync_copy(z_v, acc.at[pl.ds(r0, 16)])

            if it == 1:
                pltpu.sync_copy(vacc1.at[sl], tb_v)

                @pl.loop(0, NSCALE)
                def _(p):
                    tb_v[pl.ds(p * 16, 16)] = (tb_v[pl.ds(p * 16, 16)]
                                               * dv_v[pl.ds(p * 16, 16)])
                pltpu.sync_copy(tb_v, t2s.at[sl])

            plsc.subcore_barrier()
            if it == 1:
                pltpu.sync_copy(t2s, t_v)
            pltpu.async_copy(src_hbm.at[s, pl.ds(0, BI)], src_v.at[0],
                             bs_sem[0])
            pltpu.async_copy(dst_hbm.at[s, pl.ds(0, BI)], dst_v.at[0],
                             bd_sem[0])
        else:
            @pl.loop(0, NSCALE)
            def _(blk):
                r0 = s * SL + blk * 16
                pltpu.sync_copy(acc.at[pl.ds(r0, 16)], sbuf)
                n16 = niv_v[pl.ds(blk * 16, 16)]
                for i in range(16):
                    nii = n16[i]
                    for q in range(DH // 16):
                        sbuf[i, pl.ds(q * 16, 16)] = (
                            sbuf[i, pl.ds(q * 16, 16)] * nii)
                pltpu.sync_copy(sbuf, y_hbm.at[c, pl.ds(r0, 16)])

            @pl.when(c == 0)
            def _():
                pltpu.sync_copy(vacc1.at[sl], qv_hbm.at[sl])
                pltpu.sync_copy(vacc2.at[sl],
                                qv_hbm.at[pl.ds(N_PAD + s * SL, SL)])


BLK = 1024


def _norm_from_deg(dega, degb):
    deg = dega + degb
    return jnp.where(deg > 0.0, lax.rsqrt(jnp.maximum(deg, 1.0)), 0.0)


def _tc_first_body(x_ref, doa_ref, dob_ref, dia_ref, dib_ref,
                   ga_ref, gb_ref, no_ref, d_ref, ni_ref):
    norm_out = _norm_from_deg(doa_ref[...], dob_ref[...])
    norm_in = _norm_from_deg(dia_ref[...], dib_ref[...])
    g = x_ref[...] * norm_out
    ga_ref[...] = g[:, :DH]
    gb_ref[...] = g[:, DH:]
    no_ref[...] = norm_out
    d_ref[...] = norm_out * norm_in
    ni_ref[...] = norm_in


def _tc_first(x, doa, dob, dia, dib):
    grid = (N_PAD // BLK,)
    row = pl.BlockSpec((BLK, 1), lambda i: (i, 0))
    mat = pl.BlockSpec((BLK, D), lambda i: (i, 0))
    half = pl.BlockSpec((BLK, DH), lambda i: (i, 0))
    vec = jax.ShapeDtypeStruct((N_PAD, 1), jnp.float32)
    return pl.pallas_call(
        _tc_first_body,
        grid=grid,
        in_specs=[mat, row, row, row, row],
        out_specs=[half, half, row, row, row],
        out_shape=[jax.ShapeDtypeStruct((N_PAD, DH), jnp.float32),
                   jax.ShapeDtypeStruct((N_PAD, DH), jnp.float32),
                   vec, vec, vec],
    )(x, doa, dob, dia, dib)


def _tc_prep_body(w1_ref, w2_ref, w3_ref, b1_ref, b2_ref,
                  w123_ref, c1_ref, c2_ref):
    hi = lax.Precision.HIGHEST
    w3 = w3_ref[...]
    w12 = jnp.dot(w1_ref[...], w2_ref[...],
                  preferred_element_type=jnp.float32, precision=hi)
    w123_ref[...] = jnp.dot(w12, w3, preferred_element_type=jnp.float32,
                            precision=hi)
    b12 = jnp.dot(b1_ref[...], w2_ref[...],
                  preferred_element_type=jnp.float32, precision=hi)
    c1_ref[...] = jnp.dot(b12, w3, preferred_element_type=jnp.float32,
                          precision=hi)
    c2_ref[...] = jnp.dot(b2_ref[...], w3,
                          preferred_element_type=jnp.float32, precision=hi)


def _tc_prep(w1, w2, w3, b1r, b2r):
    wsp = pl.BlockSpec((D, D), lambda: (0, 0))
    bsp = pl.BlockSpec((1, D), lambda: (0, 0))
    return pl.pallas_call(
        _tc_prep_body,
        grid=(),
        in_specs=[wsp, wsp, wsp, bsp, bsp],
        out_specs=[wsp, bsp, bsp],
        out_shape=[jax.ShapeDtypeStruct((D, D), jnp.float32),
                   jax.ShapeDtypeStruct((1, D), jnp.float32),
                   jax.ShapeDtypeStruct((1, D), jnp.float32)],
    )(w1, w2, w3, b1r, b2r)


def _tc_final_body(sa_ref, sb_ref, wt_ref, wb_ref, c1_ref, c2_ref, b3_ref,
                   q1_ref, q2_ref, ni_ref, gam_ref, bet_ref, o_ref):
    hi = lax.Precision.HIGHEST
    m = (jnp.dot(sa_ref[...], wt_ref[...],
                 preferred_element_type=jnp.float32, precision=hi)
         + jnp.dot(sb_ref[...], wb_ref[...],
                   preferred_element_type=jnp.float32, precision=hi))
    ni = ni_ref[...]
    t = (m + (ni * q2_ref[...]) * c1_ref[...]
         + (ni * q1_ref[...]) * c2_ref[...] + b3_ref[...])
    mu = jnp.mean(t, axis=1, keepdims=True)
    cen = t - mu
    var = jnp.mean(cen * cen, axis=1, keepdims=True)
    hn = cen * lax.rsqrt(var + 1e-5)
    o_ref[...] = hn * gam_ref[...] + bet_ref[...]


def _tc_final(sa, sb, wt, wb, c1, c2, b3r, q1, q2, ni, gamma, beta):
    grid = (N_PAD // BLK,)
    row = pl.BlockSpec((BLK, 1), lambda i: (i, 0))
    half = pl.BlockSpec((BLK, DH), lambda i: (i, 0))
    mat = pl.BlockSpec((BLK, D), lambda i: (i, 0))
    wsp = pl.BlockSpec((DH, D), lambda i: (0, 0))
    bsp = pl.BlockSpec((1, D), lambda i: (0, 0))
    return pl.pallas_call(
        _tc_final_body,
        grid=grid,
        in_specs=[half, half, wsp, wsp, bsp, bsp, bsp, row, row, row,
                  bsp, bsp],
        out_specs=mat,
        out_shape=jax.ShapeDtypeStruct((N_PAD, D), jnp.float32),
    )(sa, sb, wt, wb, c1, c2, b3r, q1, q2, ni, gamma, beta)


def kernel(x, edge_index, W1, b1, W2, b2, W3, b3, gamma, beta):
    f32 = jnp.float32
    src = edge_index[0]
    dst = edge_index[1]

    pad = E_PAD - E_EDGES
    padv = jnp.full((pad,), N_NODES, jnp.int32)
    src_p = jnp.concatenate([src, padv])
    dst_p = jnp.concatenate([dst, padv])
    src3d = src_p.reshape(NUM_SUBCORES, CHUNKS, CHUNK)
    dst3d = dst_p.reshape(NUM_SUBCORES, CHUNKS, CHUNK)

    idx_deg = jnp.concatenate([src_p.reshape(NW, DCHUNKS, CHUNK),
                               dst_p.reshape(NW, DCHUNKS, CHUNK) + N_PAD],
                              axis=1)

    x_pad = jnp.concatenate([x, jnp.zeros((N_PAD - N_NODES, D), f32)])

    degp = _sc_degrees(idx_deg).reshape(NUM_CORES, 2 * N_PAD)
    doa = degp[0, :N_PAD, None]
    dob = degp[1, :N_PAD, None]
    dia = degp[0, N_PAD:, None]
    dib = degp[1, N_PAD:, None]

    ga0, gb0, no2, d2, ni2 = _tc_first(x_pad, doa, dob, dia, dib)

    b1r = b1.reshape(1, D)
    b2r = b2.reshape(1, D)
    b3r = b3.reshape(1, D)
    w123, c1, c2 = _tc_prep(W1, W2, W3, b1r, b2r)

    y, qv = _sc_gcn3(ga0, gb0, no2.reshape(-1), d2.reshape(-1),
                     ni2.reshape(-1), src3d, dst3d)

    out = _tc_final(y[0], y[1], w123[:DH], w123[DH:], c1, c2, b3r,
                    qv[:N_PAD, None], qv[N_PAD:, None], ni2,
                    gamma.reshape(1, D), beta.reshape(1, D))
    return out[:N_NODES]


# speedup vs baseline: 1.0422x; 1.0422x over previous
import functools

import jax
import jax.numpy as jnp
from jax import lax
from jax.experimental import pallas as pl
from jax.experimental.pallas import tpu as pltpu
from jax.experimental.pallas import tpu_sc as plsc

N_NODES = 10000
D = 128
DH = 64
E_EDGES = 320000

NUM_CORES = 2
NUM_SUBCORES = 16
NW = NUM_CORES * NUM_SUBCORES

CHUNK = 128
CHUNKS = 160
NBUF = 2
BI = 16
NBLOCKS = CHUNKS // BI
NGROUPS = BI // NBUF
E_PAD = NUM_SUBCORES * CHUNKS * CHUNK
DCHUNKS = E_PAD // (NW * CHUNK)
N_PAD = 10240
SL = N_PAD // NUM_SUBCORES
NSCALE = SL // 16
ZROWS = 64
NSC64 = SL // ZROWS

_mesh = plsc.VectorSubcoreMesh(
    core_axis_name="c", subcore_axis_name="s",
    num_cores=NUM_CORES, num_subcores=NUM_SUBCORES)


@functools.partial(
    pl.kernel,
    out_type=jax.ShapeDtypeStruct((NUM_CORES * 2 * N_PAD,), jnp.float32),
    mesh=_mesh,
    scratch_types=[
        pltpu.VMEM((2 * DCHUNKS, CHUNK), jnp.int32),
        pltpu.VMEM((CHUNK,), jnp.float32),
        pltpu.VMEM((2 * N_PAD // NUM_SUBCORES,), jnp.float32),
        pltpu.VMEM_SHARED((2 * N_PAD,), jnp.float32),
        pltpu.SemaphoreType.DMA,
    ],
)
def _sc_degrees(idx_hbm, out_hbm, idx_v, ones_v, z_v, acc, sem):
    c = lax.axis_index("c")
    s = lax.axis_index("s")
    wid = s * NUM_CORES + c

    ones16 = jnp.ones((16,), jnp.float32)
    zero16 = jnp.zeros((16,), jnp.float32)
    for i in range(CHUNK // 16):
        ones_v[pl.ds(i * 16, 16)] = ones16

    zlen = 2 * N_PAD // NUM_SUBCORES

    @pl.loop(0, zlen // 16)
    def _(i):
        z_v[pl.ds(i * 16, 16)] = zero16

    pltpu.sync_copy(z_v, acc.at[pl.ds(s * zlen, zlen)])
    plsc.subcore_barrier()

    pltpu.async_copy(idx_hbm.at[wid], idx_v, sem).wait()

    @pl.loop(0, 2 * DCHUNKS)
    def _(j):
        pltpu.sync_copy(ones_v, acc.at[idx_v.at[j]], add=True)

    plsc.subcore_barrier()
    pltpu.sync_copy(acc.at[pl.ds(s * zlen, zlen)],
                    out_hbm.at[pl.ds(c * 2 * N_PAD + s * zlen, zlen)])


@functools.partial(
    pl.kernel,
    out_type=[jax.ShapeDtypeStruct((NUM_CORES, N_PAD, DH), jnp.float32),
              jax.ShapeDtypeStruct((2 * N_PAD,), jnp.float32)],
    mesh=_mesh,
    scratch_types=[
        pltpu.VMEM((2, BI, CHUNK), jnp.int32),
        pltpu.VMEM((2, BI, CHUNK), jnp.int32),
        [pltpu.VMEM((CHUNK, DH), jnp.float32)] * NBUF,
        pltpu.VMEM((ZROWS, DH), jnp.float32),
        pltpu.VMEM((SL,), jnp.float32),
        pltpu.VMEM((ZROWS, DH), jnp.float32),
        pltpu.VMEM((N_PAD,), jnp.float32),
        pltpu.VMEM((SL,), jnp.float32),
        pltpu.VMEM((SL,), jnp.float32),
        pltpu.VMEM((BI, CHUNK), jnp.float32),
        pltpu.VMEM_SHARED((N_PAD, DH), jnp.float32),
        pltpu.VMEM_SHARED((N_PAD, DH), jnp.float32),
        pltpu.VMEM_SHARED((N_PAD,), jnp.float32),
        pltpu.VMEM_SHARED((N_PAD,), jnp.float32),
        pltpu.VMEM_SHARED((N_PAD,), jnp.float32),
        [pltpu.SemaphoreType.DMA] * NBUF,
        [pltpu.SemaphoreType.DMA] * NBUF,
        [pltpu.SemaphoreType.DMA] * 2,
        [pltpu.SemaphoreType.DMA] * 2,
        pltpu.SemaphoreType.DMA,
    ],
    compiler_params=pltpu.CompilerParams(use_tc_tiling_on_sc=False,
                                         needs_layout_passes=False),
)
def _sc_gcn3(ga_hbm, gb_hbm, no_hbm, d_hbm, ni_hbm, src_hbm, dst_hbm,
             y_hbm, qv_hbm,
             src_v, dst_v, rows, z_v, zv1, sbuf, t_v, dv_v, niv_v,
             sval, gtab, acc, vacc1, vacc2, t2s, gsem, ssem, bs_sem, bd_sem,
             vsem):
    tb_v = zv1
    c = lax.axis_index("c")
    s = lax.axis_index("s")
    zero16 = jnp.zeros((16,), jnp.float32)

    @pl.loop(0, ZROWS)
    def _(r):
        for i in range(DH // 16):
            z_v[r, pl.ds(i * 16, 16)] = zero16

    @pl.loop(0, NSCALE)
    def _(p):
        zv1[pl.ds(p * 16, 16)] = zero16

    sl = pl.ds(s * SL, SL)

    @pl.loop(0, SL // ZROWS)
    def _(k):
        pltpu.sync_copy(z_v, acc.at[pl.ds(s * SL + k * ZROWS, ZROWS)])

    pltpu.sync_copy(zv1, vacc1.at[sl])
    pltpu.sync_copy(zv1, vacc2.at[sl])

    @pl.when(c == 0)
    def _():
        pltpu.sync_copy(ga_hbm.at[sl], gtab.at[sl])

    @pl.when(c == 1)
    def _():
        pltpu.sync_copy(gb_hbm.at[sl], gtab.at[sl])

    pltpu.sync_copy(no_hbm, t_v)
    pltpu.sync_copy(d_hbm.at[sl], dv_v)
    pltpu.sync_copy(ni_hbm.at[sl], niv_v)

    pltpu.async_copy(src_hbm.at[s, pl.ds(0, BI)], src_v.at[0], bs_sem[0])
    pltpu.async_copy(dst_hbm.at[s, pl.ds(0, BI)], dst_v.at[0], bd_sem[0])
    plsc.subcore_barrier()

    for it in (1, 2, 3):
        vacc_it = vacc1 if it == 1 else vacc2
        for k in range(NBLOCKS):
            slot = k % 2
            pltpu.make_async_copy(src_hbm.at[s, pl.ds(k * BI, BI)],
                                  src_v.at[slot], bs_sem[slot]).wait()
            pltpu.make_async_copy(dst_hbm.at[s, pl.ds(k * BI, BI)],
                                  dst_v.at[slot], bd_sem[slot]).wait()
            if k + 1 < NBLOCKS:
                nslot = (k + 1) % 2
                pltpu.async_copy(src_hbm.at[s, pl.ds((k + 1) * BI, BI)],
                                 src_v.at[nslot], bs_sem[nslot])
                pltpu.async_copy(dst_hbm.at[s, pl.ds((k + 1) * BI, BI)],
                                 dst_v.at[nslot], bd_sem[nslot])

            for b in range(NBUF):
                pltpu.async_copy(gtab.at[src_v.at[slot, b]], rows[b],
                                 gsem[b])

            @pl.loop(0, NGROUPS)
            def _(t, slot=slot):
                base = t * NBUF
                for b in range(NBUF):
                    pltpu.make_async_copy(gtab.at[src_v.at[slot, base + b]],
                                          rows[b], gsem[b]).wait()
                    pltpu.async_copy(rows[b],
                                     acc.at[dst_v.at[slot, base + b]],
                                     ssem[b], add=True)
                for b in range(NBUF):
                    @pl.when(t + 1 < NGROUPS)
                    def _(b=b):
                        pltpu.make_async_copy(
                            rows[b], acc.at[dst_v.at[slot, base + b]],
                            ssem[b]).wait()
                        pltpu.async_copy(
                            gtab.at[src_v.at[slot, base + NBUF + b]],
                            rows[b], gsem[b])

            for b in range(NBUF):
                pltpu.make_async_copy(rows[b],
                                      acc.at[dst_v.at[slot, BI - NBUF + b]],
                                      ssem[b]).wait()

            if it < 3:
                @pl.loop(0, BI)
                def _(j, slot=slot, vacc_it=vacc_it):
                    for p in range(CHUNK // 16):
                        idx16 = src_v[slot, j, pl.ds(p * 16, 16)]
                        sval[j, pl.ds(p * 16, 16)] = plsc.load_gather(
                            t_v, [idx16])
                    pltpu.async_copy(sval.at[j],
                                     vacc_it.at[dst_v.at[slot, j]],
                                     vsem, add=True)

                @pl.loop(0, BI)
                def _(j, slot=slot, vacc_it=vacc_it):
                    pltpu.make_async_copy(sval.at[j],
                                          vacc_it.at[dst_v.at[slot, j]],
                                          vsem).wait()

        plsc.subcore_barrier()

        if it < 3:
            @pl.loop(0, NSC64)
            def _(blk):
                r0 = s * SL + blk * ZROWS
                pltpu.sync_copy(acc.at[pl.ds(r0, ZROWS)], sbuf)
                for g in range(ZROWS // 16):
                    d16 = dv_v[pl.ds(blk * ZROWS + g * 16, 16)]
                    for i in range(16):
                        di = d16[i]
                        for q in range(DH // 16):
                            sbuf[g * 16 + i, pl.ds(q * 16, 16)] = (
                                sbuf[g * 16 + i, pl.ds(q * 16, 16)] * di)
                pltpu.sync_copy(sbuf, gtab.at[pl.ds(r0, ZROWS)])
                pltpu.sync_copy(z_v, acc.at[pl.ds(r0, ZROWS)])

            if it == 1:
                pltpu.sync_copy(vacc1.at[sl], tb_v)

                @pl.loop(0, NSCALE)
                def _(p):
                    tb_v[pl.ds(p * 16, 16)] = (tb_v[pl.ds(p * 16, 16)]
                                               * dv_v[pl.ds(p * 16, 16)])
                pltpu.sync_copy(tb_v, t2s.at[sl])

            plsc.subcore_barrier()
            if it == 1:
                pltpu.sync_copy(t2s, t_v)
            pltpu.async_copy(src_hbm.at[s, pl.ds(0, BI)], src_v.at[0],
                             bs_sem[0])
            pltpu.async_copy(dst_hbm.at[s, pl.ds(0, BI)], dst_v.at[0],
                             bd_sem[0])
        else:
            @pl.loop(0, NSC64)
            def _(blk):
                r0 = s * SL + blk * ZROWS
                pltpu.sync_copy(acc.at[pl.ds(r0, ZROWS)], sbuf)
                for g in range(ZROWS // 16):
                    n16 = niv_v[pl.ds(blk * ZROWS + g * 16, 16)]
                    for i in range(16):
                        nii = n16[i]
                        for q in range(DH // 16):
                            sbuf[g * 16 + i, pl.ds(q * 16, 16)] = (
                                sbuf[g * 16 + i, pl.ds(q * 16, 16)] * nii)
                pltpu.sync_copy(sbuf, y_hbm.at[c, pl.ds(r0, ZROWS)])

            @pl.when(c == 0)
            def _():
                pltpu.sync_copy(vacc1.at[sl], qv_hbm.at[sl])
                pltpu.sync_copy(vacc2.at[sl],
                                qv_hbm.at[pl.ds(N_PAD + s * SL, SL)])


BLK = 1024


def _norm_from_deg(dega, degb):
    deg = dega + degb
    return jnp.where(deg > 0.0, lax.rsqrt(jnp.maximum(deg, 1.0)), 0.0)


def _tc_first_body(x_ref, doa_ref, dob_ref, dia_ref, dib_ref,
                   ga_ref, gb_ref, no_ref, d_ref, ni_ref):
    norm_out = _norm_from_deg(doa_ref[...], dob_ref[...])
    norm_in = _norm_from_deg(dia_ref[...], dib_ref[...])
    g = x_ref[...] * norm_out
    ga_ref[...] = g[:, :DH]
    gb_ref[...] = g[:, DH:]
    no_ref[...] = norm_out
    d_ref[...] = norm_out * norm_in
    ni_ref[...] = norm_in


def _tc_first(x, doa, dob, dia, dib):
    grid = (N_PAD // BLK,)
    row = pl.BlockSpec((BLK, 1), lambda i: (i, 0))
    mat = pl.BlockSpec((BLK, D), lambda i: (i, 0))
    half = pl.BlockSpec((BLK, DH), lambda i: (i, 0))
    vec = jax.ShapeDtypeStruct((N_PAD, 1), jnp.float32)
    return pl.pallas_call(
        _tc_first_body,
        grid=grid,
        in_specs=[mat, row, row, row, row],
        out_specs=[half, half, row, row, row],
        out_shape=[jax.ShapeDtypeStruct((N_PAD, DH), jnp.float32),
                   jax.ShapeDtypeStruct((N_PAD, DH), jnp.float32),
                   vec, vec, vec],
    )(x, doa, dob, dia, dib)


def _tc_prep_body(w1_ref, w2_ref, w3_ref, b1_ref, b2_ref,
                  w123_ref, c1_ref, c2_ref):
    hi = lax.Precision.HIGHEST
    w3 = w3_ref[...]
    w12 = jnp.dot(w1_ref[...], w2_ref[...],
                  preferred_element_type=jnp.float32, precision=hi)
    w123_ref[...] = jnp.dot(w12, w3, preferred_element_type=jnp.float32,
                            precision=hi)
    b12 = jnp.dot(b1_ref[...], w2_ref[...],
                  preferred_element_type=jnp.float32, precision=hi)
    c1_ref[...] = jnp.dot(b12, w3, preferred_element_type=jnp.float32,
                          precision=hi)
    c2_ref[...] = jnp.dot(b2_ref[...], w3,
                          preferred_element_type=jnp.float32, precision=hi)


def _tc_prep(w1, w2, w3, b1r, b2r):
    wsp = pl.BlockSpec((D, D), lambda: (0, 0))
    bsp = pl.BlockSpec((1, D), lambda: (0, 0))
    return pl.pallas_call(
        _tc_prep_body,
        grid=(),
        in_specs=[wsp, wsp, wsp, bsp, bsp],
        out_specs=[wsp, bsp, bsp],
        out_shape=[jax.ShapeDtypeStruct((D, D), jnp.float32),
                   jax.ShapeDtypeStruct((1, D), jnp.float32),
                   jax.ShapeDtypeStruct((1, D), jnp.float32)],
    )(w1, w2, w3, b1r, b2r)


def _tc_final_body(sa_ref, sb_ref, wt_ref, wb_ref, c1_ref, c2_ref, b3_ref,
                   q1_ref, q2_ref, ni_ref, gam_ref, bet_ref, o_ref):
    hi = lax.Precision.HIGHEST
    m = (jnp.dot(sa_ref[...], wt_ref[...],
                 preferred_element_type=jnp.float32, precision=hi)
         + jnp.dot(sb_ref[...], wb_ref[...],
                   preferred_element_type=jnp.float32, precision=hi))
    ni = ni_ref[...]
    t = (m + (ni * q2_ref[...]) * c1_ref[...]
         + (ni * q1_ref[...]) * c2_ref[...] + b3_ref[...])
    mu = jnp.mean(t, axis=1, keepdims=True)
    cen = t - mu
    var = jnp.mean(cen * cen, axis=1, keepdims=True)
    hn = cen * lax.rsqrt(var + 1e-5)
    o_ref[...] = hn * gam_ref[...] + bet_ref[...]


def _tc_final(sa, sb, wt, wb, c1, c2, b3r, q1, q2, ni, gamma, beta):
    grid = (N_PAD // BLK,)
    row = pl.BlockSpec((BLK, 1), lambda i: (i, 0))
    half = pl.BlockSpec((BLK, DH), lambda i: (i, 0))
    mat = pl.BlockSpec((BLK, D), lambda i: (i, 0))
    wsp = pl.BlockSpec((DH, D), lambda i: (0, 0))
    bsp = pl.BlockSpec((1, D), lambda i: (0, 0))
    return pl.pallas_call(
        _tc_final_body,
        grid=grid,
        in_specs=[half, half, wsp, wsp, bsp, bsp, bsp, row, row, row,
                  bsp, bsp],
        out_specs=mat,
        out_shape=jax.ShapeDtypeStruct((N_PAD, D), jnp.float32),
    )(sa, sb, wt, wb, c1, c2, b3r, q1, q2, ni, gamma, beta)


def kernel(x, edge_index, W1, b1, W2, b2, W3, b3, gamma, beta):
    f32 = jnp.float32
    src = edge_index[0]
    dst = edge_index[1]

    pad = E_PAD - E_EDGES
    padv = jnp.full((pad,), N_NODES, jnp.int32)
    src_p = jnp.concatenate([src, padv])
    dst_p = jnp.concatenate([dst, padv])
    src3d = src_p.reshape(NUM_SUBCORES, CHUNKS, CHUNK)
    dst3d = dst_p.reshape(NUM_SUBCORES, CHUNKS, CHUNK)

    idx_deg = jnp.concatenate([src_p.reshape(NW, DCHUNKS, CHUNK),
                               dst_p.reshape(NW, DCHUNKS, CHUNK) + N_PAD],
                              axis=1)

    x_pad = jnp.concatenate([x, jnp.zeros((N_PAD - N_NODES, D), f32)])

    degp = _sc_degrees(idx_deg).reshape(NUM_CORES, 2 * N_PAD)
    doa = degp[0, :N_PAD, None]
    dob = degp[1, :N_PAD, None]
    dia = degp[0, N_PAD:, None]
    dib = degp[1, N_PAD:, None]

    ga0, gb0, no2, d2, ni2 = _tc_first(x_pad, doa, dob, dia, dib)

    b1r = b1.reshape(1, D)
    b2r = b2.reshape(1, D)
    b3r = b3.reshape(1, D)
    w123, c1, c2 = _tc_prep(W1, W2, W3, b1r, b2r)

    y, qv = _sc_gcn3(ga0, gb0, no2.reshape(-1), d2.reshape(-1),
                     ni2.reshape(-1), src3d, dst3d)

    out = _tc_final(y[0], y[1], w123[:DH], w123[DH:], c1, c2, b3r,
                    qv[:N_PAD, None], qv[N_PAD:, None], ni2,
                    gamma.reshape(1, D), beta.reshape(1, D))
    return out[:N_NODES]


# speedup vs baseline: 1.1261x; 1.0804x over previous
import functools

import jax
import jax.numpy as jnp
from jax import lax
from jax.experimental import pallas as pl
from jax.experimental.pallas import tpu as pltpu
from jax.experimental.pallas import tpu_sc as plsc

N_NODES = 10000
D = 128
DH = 64
E_EDGES = 320000

NUM_CORES = 2
NUM_SUBCORES = 16
NW = NUM_CORES * NUM_SUBCORES

CHUNK = 128
CHUNKS = 160
NBUF = 2
BI = 40
NBLOCKS = CHUNKS // BI
NGROUPS = BI // NBUF
E_PAD = NUM_SUBCORES * CHUNKS * CHUNK
DCHUNKS = E_PAD // (NW * CHUNK)
N_PAD = 10240
ROWS_PER_TILE = N_PAD // NUM_SUBCORES
ZROWS = 16

_mesh = plsc.VectorSubcoreMesh(
    core_axis_name="c", subcore_axis_name="s",
    num_cores=NUM_CORES, num_subcores=NUM_SUBCORES)


@functools.partial(
    pl.kernel,
    out_type=jax.ShapeDtypeStruct((NUM_CORES * 2 * N_PAD,), jnp.float32),
    mesh=_mesh,
    scratch_types=[
        pltpu.VMEM((2 * DCHUNKS, CHUNK), jnp.int32),
        pltpu.VMEM((CHUNK,), jnp.float32),
        pltpu.VMEM((2 * N_PAD // NUM_SUBCORES,), jnp.float32),
        pltpu.VMEM_SHARED((2 * N_PAD,), jnp.float32),
        pltpu.SemaphoreType.DMA,
    ],
)
def _sc_degrees(idx_hbm, out_hbm, idx_v, ones_v, z_v, acc, sem):
    c = lax.axis_index("c")
    s = lax.axis_index("s")
    wid = s * NUM_CORES + c

    ones16 = jnp.ones((16,), jnp.float32)
    zero16 = jnp.zeros((16,), jnp.float32)
    for i in range(CHUNK // 16):
        ones_v[pl.ds(i * 16, 16)] = ones16

    zlen = 2 * N_PAD // NUM_SUBCORES

    @pl.loop(0, zlen // 16)
    def _(i):
        z_v[pl.ds(i * 16, 16)] = zero16

    pltpu.sync_copy(z_v, acc.at[pl.ds(s * zlen, zlen)])
    plsc.subcore_barrier()

    pltpu.async_copy(idx_hbm.at[wid], idx_v, sem).wait()

    @pl.loop(0, 2 * DCHUNKS)
    def _(j):
        pltpu.sync_copy(ones_v, acc.at[idx_v.at[j]], add=True)

    plsc.subcore_barrier()
    pltpu.sync_copy(acc.at[pl.ds(s * zlen, zlen)],
                    out_hbm.at[pl.ds(c * 2 * N_PAD + s * zlen, zlen)])


@functools.partial(
    pl.kernel,
    out_type=jax.ShapeDtypeStruct((NUM_CORES, N_PAD, DH), jnp.float32),
    mesh=_mesh,
    scratch_types=[
        pltpu.VMEM((2, BI, CHUNK), jnp.int32),
        pltpu.VMEM((2, BI, CHUNK), jnp.int32),
        [pltpu.VMEM((CHUNK, DH), jnp.float32)] * NBUF,
        pltpu.VMEM((ZROWS, DH), jnp.float32),
        pltpu.VMEM_SHARED((N_PAD, DH), jnp.float32),
        pltpu.VMEM_SHARED((N_PAD, DH), jnp.float32),
        [pltpu.SemaphoreType.DMA] * NBUF,
        [pltpu.SemaphoreType.DMA] * NBUF,
        [pltpu.SemaphoreType.DMA] * 2,
        [pltpu.SemaphoreType.DMA] * 2,
    ],
    compiler_params=pltpu.CompilerParams(use_tc_tiling_on_sc=False),
)
def _sc_aggregate(ga_hbm, gb_hbm, src_hbm, dst_hbm, out_hbm,
                  src_v, dst_v, rows, z_v, gtab, acc, gsem, ssem, bs_sem,
                  bd_sem):
    c = lax.axis_index("c")
    s = lax.axis_index("s")

    zero16 = jnp.zeros((16,), jnp.float32)

    @pl.loop(0, ZROWS)
    def _(r):
        for i in range(DH // 16):
            z_v[r, pl.ds(i * 16, 16)] = zero16

    @pl.loop(0, ROWS_PER_TILE // ZROWS)
    def _(k):
        pltpu.sync_copy(z_v, acc.at[pl.ds(s * ROWS_PER_TILE + k * ZROWS,
                                          ZROWS)])

    rslice = pl.ds(s * ROWS_PER_TILE, ROWS_PER_TILE)

    @pl.when(c == 0)
    def _():
        pltpu.sync_copy(ga_hbm.at[rslice], gtab.at[rslice])

    @pl.when(c == 1)
    def _():
        pltpu.sync_copy(gb_hbm.at[rslice], gtab.at[rslice])

    pltpu.async_copy(src_hbm.at[s, pl.ds(0, BI)], src_v.at[0], bs_sem[0])
    pltpu.async_copy(dst_hbm.at[s, pl.ds(0, BI)], dst_v.at[0], bd_sem[0])
    plsc.subcore_barrier()

    for k in range(NBLOCKS):
        slot = k % 2
        pltpu.make_async_copy(src_hbm.at[s, pl.ds(k * BI, BI)],
                              src_v.at[slot], bs_sem[slot]).wait()
        pltpu.make_async_copy(dst_hbm.at[s, pl.ds(k * BI, BI)],
                              dst_v.at[slot], bd_sem[slot]).wait()
        if k + 1 < NBLOCKS:
            nslot = (k + 1) % 2
            pltpu.async_copy(src_hbm.at[s, pl.ds((k + 1) * BI, BI)],
                             src_v.at[nslot], bs_sem[nslot])
            pltpu.async_copy(dst_hbm.at[s, pl.ds((k + 1) * BI, BI)],
                             dst_v.at[nslot], bd_sem[nslot])

        for b in range(NBUF):
            pltpu.async_copy(gtab.at[src_v.at[slot, b]], rows[b], gsem[b])

        @pl.loop(0, NGROUPS)
        def _(t, slot=slot):
            base = t * NBUF
            for b in range(NBUF):
                pltpu.make_async_copy(gtab.at[src_v.at[slot, base + b]],
                                      rows[b], gsem[b]).wait()
                pltpu.async_copy(rows[b], acc.at[dst_v.at[slot, base + b]],
                                 ssem[b], add=True)
            for b in range(NBUF):
                @pl.when(t + 1 < NGROUPS)
                def _(b=b):
                    pltpu.make_async_copy(rows[b],
                                          acc.at[dst_v.at[slot, base + b]],
                                          ssem[b]).wait()
                    pltpu.async_copy(gtab.at[src_v.at[slot, base + NBUF + b]],
                                     rows[b], gsem[b])

        for b in range(NBUF):
            pltpu.make_async_copy(rows[b],
                                  acc.at[dst_v.at[slot, BI - NBUF + b]],
                                  ssem[b]).wait()

    plsc.subcore_barrier()
    pltpu.sync_copy(acc.at[pl.ds(s * ROWS_PER_TILE, ROWS_PER_TILE)],
                    out_hbm.at[c, pl.ds(s * ROWS_PER_TILE, ROWS_PER_TILE)])


BLK = 1024


def _norm_from_deg(dega, degb):
    deg = dega + degb
    return jnp.where(deg > 0.0, lax.rsqrt(jnp.maximum(deg, 1.0)), 0.0)


def _tc_first_body(x_ref, doa_ref, dob_ref, dia_ref, dib_ref,
                   ga_ref, gb_ref, no_ref, ni_ref):
    norm_out = _norm_from_deg(doa_ref[...], dob_ref[...])
    norm_in = _norm_from_deg(dia_ref[...], dib_ref[...])
    g = x_ref[...] * norm_out
    ga_ref[...] = g[:, :DH]
    gb_ref[...] = g[:, DH:]
    no_ref[...] = norm_out
    ni_ref[...] = norm_in


def _tc_first(x, doa, dob, dia, dib):
    grid = (N_PAD // BLK,)
    row = pl.BlockSpec((BLK, 1), lambda i: (i, 0))
    mat = pl.BlockSpec((BLK, D), lambda i: (i, 0))
    half = pl.BlockSpec((BLK, DH), lambda i: (i, 0))
    return pl.pallas_call(
        _tc_first_body,
        grid=grid,
        in_specs=[mat, row, row, row, row],
        out_specs=[half, half, row, row],
        out_shape=[jax.ShapeDtypeStruct((N_PAD, DH), jnp.float32),
                   jax.ShapeDtypeStruct((N_PAD, DH), jnp.float32),
                   jax.ShapeDtypeStruct((N_PAD, 1), jnp.float32),
                   jax.ShapeDtypeStruct((N_PAD, 1), jnp.float32)],
    )(x, doa, dob, dia, dib)


def _tc_mid_body(sa_ref, sb_ref, wt_ref, wb_ref, b_ref, no_ref, ni_ref,
                 ga_ref, gb_ref):
    m = (jnp.dot(sa_ref[...], wt_ref[...],
                 preferred_element_type=jnp.float32,
                 precision=lax.Precision.HIGHEST)
         + jnp.dot(sb_ref[...], wb_ref[...],
                   preferred_element_type=jnp.float32,
                   precision=lax.Precision.HIGHEST))
    no = no_ref[...]
    g = (no * ni_ref[...]) * m + no * b_ref[...]
    ga_ref[...] = g[:, :DH]
    gb_ref[...] = g[:, DH:]


def _tc_mid(sa, sb, wt, wb, b, no, ni):
    grid = (N_PAD // BLK,)
    row = pl.BlockSpec((BLK, 1), lambda i: (i, 0))
    half = pl.BlockSpec((BLK, DH), lambda i: (i, 0))
    wsp = pl.BlockSpec((DH, D), lambda i: (0, 0))
    bsp = pl.BlockSpec((1, D), lambda i: (0, 0))
    return pl.pallas_call(
        _tc_mid_body,
        grid=grid,
        in_specs=[half, half, wsp, wsp, bsp, row, row],
        out_specs=[half, half],
        out_shape=[jax.ShapeDtypeStruct((N_PAD, DH), jnp.float32),
                   jax.ShapeDtypeStruct((N_PAD, DH), jnp.float32)],
    )(sa, sb, wt, wb, b, no, ni)


def _tc_final_body(sa_ref, sb_ref, wt_ref, wb_ref, b_ref, ni_ref, gam_ref,
                   bet_ref, o_ref):
    m = (jnp.dot(sa_ref[...], wt_ref[...],
                 preferred_element_type=jnp.float32,
                 precision=lax.Precision.HIGHEST)
         + jnp.dot(sb_ref[...], wb_ref[...],
                   preferred_element_type=jnp.float32,
                   precision=lax.Precision.HIGHEST))
    t = ni_ref[...] * m + b_ref[...]
    mu = jnp.mean(t, axis=1, keepdims=True)
    cen = t - mu
    var = jnp.mean(cen * cen, axis=1, keepdims=True)
    hn = cen * lax.rsqrt(var + 1e-5)
    o_ref[...] = hn * gam_ref[...] + bet_ref[...]


def _tc_final(sa, sb, wt, wb, b, ni, gamma, beta):
    grid = (N_PAD // BLK,)
    row = pl.BlockSpec((BLK, 1), lambda i: (i, 0))
    half = pl.BlockSpec((BLK, DH), lambda i: (i, 0))
    mat = pl.BlockSpec((BLK, D), lambda i: (i, 0))
    wsp = pl.BlockSpec((DH, D), lambda i: (0, 0))
    bsp = pl.BlockSpec((1, D), lambda i: (0, 0))
    return pl.pallas_call(
        _tc_final_body,
        grid=grid,
        in_specs=[half, half, wsp, wsp, bsp, row, bsp, bsp],
        out_specs=mat,
        out_shape=jax.ShapeDtypeStruct((N_PAD, D), jnp.float32),
    )(sa, sb, wt, wb, b, ni, gamma, beta)


def kernel(x, edge_index, W1, b1, W2, b2, W3, b3, gamma, beta):
    f32 = jnp.float32
    src = edge_index[0]
    dst = edge_index[1]

    pad = E_PAD - E_EDGES
    padv = jnp.full((pad,), N_NODES, jnp.int32)
    src_p = jnp.concatenate([src, padv])
    dst_p = jnp.concatenate([dst, padv])
    src3d = src_p.reshape(NUM_SUBCORES, CHUNKS, CHUNK)
    dst3d = dst_p.reshape(NUM_SUBCORES, CHUNKS, CHUNK)

    idx_deg = jnp.concatenate([src_p.reshape(NW, DCHUNKS, CHUNK),
                               dst_p.reshape(NW, DCHUNKS, CHUNK) + N_PAD],
                              axis=1)

    x_pad = jnp.concatenate([x, jnp.zeros((N_PAD - N_NODES, D), f32)])

    degp = _sc_degrees(idx_deg).reshape(NUM_CORES, 2 * N_PAD)
    doa = degp[0, :N_PAD, None]
    dob = degp[1, :N_PAD, None]
    dia = degp[0, N_PAD:, None]
    dib = degp[1, N_PAD:, None]

    ga0, gb0, no, ni = _tc_first(x_pad, doa, dob, dia, dib)

    b1r = b1.reshape(1, D)
    b2r = b2.reshape(1, D)
    b3r = b3.reshape(1, D)

    s1 = _sc_aggregate(ga0, gb0, src3d, dst3d)
    ga1, gb1 = _tc_mid(s1[0], s1[1], W1[:DH], W1[DH:], b1r, no, ni)
    s2 = _sc_aggregate(ga1, gb1, src3d, dst3d)
    ga2, gb2 = _tc_mid(s2[0], s2[1], W2[:DH], W2[DH:], b2r, no, ni)
    s3 = _sc_aggregate(ga2, gb2, src3d, dst3d)
    out = _tc_final(s3[0], s3[1], W3[:DH], W3[DH:], b3r, ni,
                    gamma.reshape(1, D), beta.reshape(1, D))
    return out[:N_NODES]
